# baseline (speedup 1.0000x reference)
import functools
import math

import jax
import jax.numpy as jnp
import numpy as np
from jax import lax
from jax.experimental import pallas as pl
from jax.experimental.pallas import tpu as pltpu
from jax.experimental.pallas import tpu_sc as plsc


def _silu(x):
    return x * (1.0 / (1.0 + jnp.exp(-x)))


def _node_fctp_body(scale, x_ref, y_ref, deg_ref, wi_ref, wm_ref, nf_ref, mask_ref):
    x = x_ref[...]
    y = y_ref[...]
    bn, d = x.shape
    a = y.shape[1]
    accf = jnp.zeros((bn, d), jnp.float32)
    accm = jnp.zeros((bn, d), jnp.float32)
    for j in range(a):
        xy = x * y[:, j:j + 1]
        accf += jnp.dot(xy, wi_ref[j], preferred_element_type=jnp.float32)
        accm += jnp.dot(xy, wm_ref[j], preferred_element_type=jnp.float32)
    nf_ref[...] = accf * scale * lax.rsqrt(deg_ref[...])
    mask_ref[...] = accm * scale


def _edge_mlp_body(s0, s1, s2, de, elen_ref, attr_ref, w0_ref, w1_ref, w2_ref, s_ref):
    bf = jnp.bfloat16
    h = _silu(jnp.dot(elen_ref[...].astype(bf), w0_ref[...],
                      preferred_element_type=jnp.float32) * s0)
    h = _silu(jnp.dot(h.astype(bf), w1_ref[...],
                      preferred_element_type=jnp.float32) * s1)
    hb = h.astype(bf)
    attr = attr_ref[...].astype(bf)
    z = jnp.concatenate([hb * attr[:, v:v + 1] for v in range(de)], axis=1)
    s_ref[...] = jnp.dot(z, w2_ref[...], preferred_element_type=jnp.float32) * s2


def _out_fctp_body(scale, c_s, c_x, aggp_ref, y_ref, deg_ref, mask_ref, wo_ref, out_ref):
    agg = aggp_ref[0] * lax.rsqrt(deg_ref[...])
    y = y_ref[...]
    bn, d = agg.shape
    a = y.shape[1]
    acc = jnp.zeros((bn, d), jnp.float32)
    for j in range(a):
        acc += jnp.dot(agg * y[:, j:j + 1], wo_ref[j], preferred_element_type=jnp.float32)
    out_ref[...] = c_s * mask_ref[...] + c_x * scale * acc


def _sc_edge_body(N, D, CH, NCH, NSP, NC, H,
                  nf_hbm, s_hbm, src_hbm, dst_hbm, out_hbm,
                  src_v, dst_v, x_v, s2_v, ef_v, dstl_v, agg_sh, sem):
    c = lax.axis_index("c")
    sid = lax.axis_index("s")
    lo = c * H

    zrows = NSP // 16
    zero16 = jnp.zeros((16,), jnp.float32)

    def zrow(r, carry):
        for cc in range(D // 16):
            ef_v[r, pl.ds(cc * 16, 16)] = zero16
        return carry

    lax.fori_loop(0, CH, zrow, 0)
    nzc, rem = zrows // CH, zrows % CH
    for t in range(nzc):
        pltpu.sync_copy(ef_v, agg_sh.at[pl.ds(sid * zrows + t * CH, CH)])
    if rem:
        pltpu.sync_copy(ef_v.at[pl.ds(0, rem)],
                        agg_sh.at[pl.ds(sid * zrows + nzc * CH, rem)])
    plsc.subcore_barrier()

    ept = NCH * CH
    base = sid * ept
    NPH = NCH // 2

    for ph in range(2):
        pltpu.sync_copy(src_hbm.at[sid].at[pl.ds(ph * NPH, NPH)], src_v)
        pltpu.sync_copy(dst_hbm.at[sid].at[pl.ds(ph * NPH, NPH)], dst_v)

        def pair(t, carry):
            jg = (ph * NPH) + 2 * t
            pltpu.sync_copy(s_hbm.at[pl.ds(base + jg * CH, 2 * CH)], s2_v)
            for u in range(2):
                jloc = 2 * t + u
                pltpu.async_copy(nf_hbm.at[src_v.at[jloc]], x_v, sem).wait()

                def mrow(r, cr):
                    for cc in range(D // 16):
                        sl = pl.ds(cc * 16, 16)
                        ef_v[r, sl] = s2_v[u * CH + r, sl] * x_v[r, sl]
                    return cr

                lax.fori_loop(0, CH, mrow, 0)
                for cc in range(CH // 16):
                    sl = pl.ds(cc * 16, 16)
                    loc = dst_v[jloc, sl] - lo
                    ok = (loc >= 0) & (loc < H)
                    dstl_v[0, sl] = jnp.where(ok, loc, H)
                pltpu.sync_copy(ef_v, agg_sh.at[dstl_v.at[0]], add=True)
            return carry

        lax.fori_loop(0, NPH // 2, pair, 0)
    plsc.subcore_barrier()

    orows = NSP // 16
    pltpu.sync_copy(agg_sh.at[pl.ds(sid * orows, orows)],
                    out_hbm.at[c].at[pl.ds(sid * orows, orows)])


def kernel(node_input, node_attr, node_deg, edge_src, edge_dst, edge_attr,
           edge_length_embedded, numb, n, W_in, W_mask, W_out, W_e0, W_e1, W_e2):
    N, D = node_input.shape
    A = node_attr.shape[1]
    E = edge_src.shape[0]
    DE = edge_attr.shape[1]
    NB = edge_length_embedded.shape[1]
    RN = W_e1.shape[0]

    NS = 16
    NC = 2
    CH = 128
    H = N // NC
    NCH = (E + NS * CH - 1) // (NS * CH)
    NCH = ((NCH + 3) // 4) * 4
    ept = NCH * CH
    E_pad = NS * ept
    NSP = ((H + 1 + NS * 8 - 1) // (NS * 8)) * NS * 8

    fctp_scale = 1.0 / math.sqrt(D * A)
    c_s, c_x = math.sin(math.pi / 8), math.cos(math.pi / 8)

    pad = E_pad - E
    src_p = jnp.pad(edge_src.astype(jnp.int32), (0, pad)).reshape(NS, NCH, CH)
    dst_p = jnp.pad(edge_dst.astype(jnp.int32), (0, pad),
                    constant_values=N).reshape(NS, NCH, CH)
    W_in_r = jnp.transpose(W_in, (1, 0, 2))
    W_mask_r = jnp.transpose(W_mask, (1, 0, 2))
    W_out_r = jnp.transpose(W_out, (1, 0, 2))
    W2z = jnp.transpose(W_e2.reshape(RN, D, DE), (2, 0, 1)).reshape(DE * RN, D)
    W_e0b = W_e0.astype(jnp.bfloat16)
    W_e1b = W_e1.astype(jnp.bfloat16)
    W2zb = W2z.astype(jnp.bfloat16)

    BN = 1000
    grid_n = N // BN
    nf, mask = pl.pallas_call(
        functools.partial(_node_fctp_body, fctp_scale),
        grid=(grid_n,),
        in_specs=[
            pl.BlockSpec((BN, D), lambda i: (i, 0)),
            pl.BlockSpec((BN, A), lambda i: (i, 0)),
            pl.BlockSpec((BN, 1), lambda i: (i, 0)),
            pl.BlockSpec((A, D, D), lambda i: (0, 0, 0)),
            pl.BlockSpec((A, D, D), lambda i: (0, 0, 0)),
        ],
        out_specs=[
            pl.BlockSpec((BN, D), lambda i: (i, 0)),
            pl.BlockSpec((BN, D), lambda i: (i, 0)),
        ],
        out_shape=[
            jax.ShapeDtypeStruct((N, D), jnp.float32),
            jax.ShapeDtypeStruct((N, D), jnp.float32),
        ],
    )(node_input, node_attr, node_deg, W_in_r, W_mask_r)

    BE = 2048
    grid_e = E_pad // BE
    LB = (E + BE - 1) // BE - 1
    s0 = 1.0 / math.sqrt(NB)
    s1 = 1.0 / math.sqrt(RN)
    s2 = 1.0 / (math.sqrt(RN) * math.sqrt(DE))
    s_dense = pl.pallas_call(
        functools.partial(_edge_mlp_body, s0, s1, s2, DE),
        grid=(grid_e,),
        in_specs=[
            pl.BlockSpec((BE, NB), lambda i: (jnp.minimum(i, LB), 0)),
            pl.BlockSpec((BE, DE), lambda i: (jnp.minimum(i, LB), 0)),
            pl.BlockSpec((NB, RN), lambda i: (0, 0)),
            pl.BlockSpec((RN, RN), lambda i: (0, 0)),
            pl.BlockSpec((DE * RN, D), lambda i: (0, 0)),
        ],
        out_specs=pl.BlockSpec((BE, D), lambda i: (i, 0)),
        out_shape=jax.ShapeDtypeStruct((E_pad, D), jnp.float32),
    )(edge_length_embedded, edge_attr, W_e0b, W_e1b, W2zb)

    sc_edge = functools.partial(
        pl.kernel,
        mesh=plsc.VectorSubcoreMesh(core_axis_name="c", subcore_axis_name="s"),
        out_type=jax.ShapeDtypeStruct((NC, NSP, D), jnp.float32),
        scratch_types=[
            pltpu.VMEM((NCH // 2, CH), jnp.int32),
            pltpu.VMEM((NCH // 2, CH), jnp.int32),
            pltpu.VMEM((CH, D), jnp.float32),
            pltpu.VMEM((2 * CH, D), jnp.float32),
            pltpu.VMEM((CH, D), jnp.float32),
            pltpu.VMEM((1, CH), jnp.int32),
            pltpu.VMEM_SHARED((NSP, D), jnp.float32),
        ] + [pltpu.SemaphoreType.DMA],
    )(functools.partial(_sc_edge_body, N, D, CH, NCH, NSP, NC, H))
    aggp = sc_edge(nf, s_dense, src_p, dst_p)

    out = pl.pallas_call(
        functools.partial(_out_fctp_body, fctp_scale, c_s, c_x),
        grid=(grid_n,),
        in_specs=[
            pl.BlockSpec((1, BN, D), lambda i: (i // (H // BN), i % (H // BN), 0)),
            pl.BlockSpec((BN, A), lambda i: (i, 0)),
            pl.BlockSpec((BN, 1), lambda i: (i, 0)),
            pl.BlockSpec((BN, D), lambda i: (i, 0)),
            pl.BlockSpec((A, D, D), lambda i: (0, 0, 0)),
        ],
        out_specs=pl.BlockSpec((BN, D), lambda i: (i, 0)),
        out_shape=jax.ShapeDtypeStruct((N, D), jnp.float32),
    )(aggp, node_attr, node_deg, mask, W_out_r)
    return out


# speedup vs baseline: 1.2699x; 1.2699x over previous
import functools
import math

import jax
import jax.numpy as jnp
from jax import lax
from jax.experimental import pallas as pl
from jax.experimental.pallas import tpu as pltpu
from jax.experimental.pallas import tpu_sc as plsc


def _silu(x):
    return x * (1.0 / (1.0 + jnp.exp(-x)))


def _node_fctp_body(scale, x_ref, y_ref, deg_ref, wi_ref, wm_ref, nf_ref, mask_ref):
    x = x_ref[...]
    y = y_ref[...]
    bn, d = x.shape
    a = y.shape[1]
    accf = jnp.zeros((bn, d), jnp.float32)
    accm = jnp.zeros((bn, d), jnp.float32)
    for j in range(a):
        xy = x * y[:, j:j + 1]
        accf += jnp.dot(xy, wi_ref[j], preferred_element_type=jnp.float32)
        accm += jnp.dot(xy, wm_ref[j], preferred_element_type=jnp.float32)
    nf_ref[...] = accf * scale * lax.rsqrt(deg_ref[...])
    mask_ref[...] = accm * scale


def _edge_mlp_body(s0, s1, s2, de, elen_ref, attr_ref, w0_ref, w1_ref, w2_ref, s_ref):
    bf = jnp.bfloat16
    h = _silu(jnp.dot(elen_ref[...].astype(bf), w0_ref[...],
                      preferred_element_type=jnp.float32) * s0)
    h = _silu(jnp.dot(h.astype(bf), w1_ref[...],
                      preferred_element_type=jnp.float32) * s1)
    hb = h.astype(bf)
    attr = attr_ref[...].astype(bf)
    z = jnp.concatenate([hb * attr[:, v:v + 1] for v in range(de)], axis=1)
    s_ref[...] = jnp.dot(z, w2_ref[...], preferred_element_type=jnp.float32) * s2


def _out_fctp_body(scale, c_s, c_x, aggp_ref, y_ref, deg_ref, mask_ref, wo_ref, out_ref):
    agg = aggp_ref[0] * lax.rsqrt(deg_ref[...])
    y = y_ref[...]
    bn, d = agg.shape
    a = y.shape[1]
    acc = jnp.zeros((bn, d), jnp.float32)
    for j in range(a):
        acc += jnp.dot(agg * y[:, j:j + 1], wo_ref[j], preferred_element_type=jnp.float32)
    out_ref[...] = c_s * mask_ref[...] + c_x * scale * acc


def _sc_edge_body(N, D, CH, NCH, NSP, NC, H,
                  nf_hbm, s_hbm, src_hbm, dst_hbm, out_hbm,
                  src_v, dst_v, x_v, ef_v, dstl_v, agg_sh, sem):
    c = lax.axis_index("c")
    sid = lax.axis_index("s")
    lo = c * H

    zrows = NSP // 16
    zero16 = jnp.zeros((16,), jnp.float32)

    def zrow(r, carry):
        for cc in range(D // 16):
            ef_v[r, pl.ds(cc * 16, 16)] = zero16
        return carry

    lax.fori_loop(0, CH, zrow, 0)
    nzc, rem = zrows // CH, zrows % CH
    for t in range(nzc):
        pltpu.sync_copy(ef_v, agg_sh.at[pl.ds(sid * zrows + t * CH, CH)])
    if rem:
        pltpu.sync_copy(ef_v.at[pl.ds(0, rem)],
                        agg_sh.at[pl.ds(sid * zrows + nzc * CH, rem)])
    plsc.subcore_barrier()

    pltpu.sync_copy(src_hbm.at[sid], src_v)
    pltpu.sync_copy(dst_hbm.at[sid], dst_v)

    ept = NCH * CH
    base = sid * ept

    def chunk(j, carry):
        pltpu.async_copy(nf_hbm.at[src_v.at[j]], x_v, sem).wait()
        pltpu.sync_copy(s_hbm.at[pl.ds(base + j * CH, CH)], ef_v)

        def mrow(r, cr):
            for cc in range(D // 16):
                sl = pl.ds(cc * 16, 16)
                ef_v[r, sl] = ef_v[r, sl] * x_v[r, sl]
            return cr

        lax.fori_loop(0, CH, mrow, 0)
        for cc in range(CH // 16):
            sl = pl.ds(cc * 16, 16)
            loc = dst_v[j, sl] - lo
            ok = (loc >= 0) & (loc < H)
            dstl_v[0, sl] = jnp.where(ok, loc, H)
        pltpu.sync_copy(ef_v, agg_sh.at[dstl_v.at[0]], add=True)
        return carry

    lax.fori_loop(0, NCH, chunk, 0)
    plsc.subcore_barrier()

    orows = NSP // 16
    pltpu.sync_copy(agg_sh.at[pl.ds(sid * orows, orows)],
                    out_hbm.at[c].at[pl.ds(sid * orows, orows)])


def kernel(node_input, node_attr, node_deg, edge_src, edge_dst, edge_attr,
           edge_length_embedded, numb, n, W_in, W_mask, W_out, W_e0, W_e1, W_e2):
    N, D = node_input.shape
    A = node_attr.shape[1]
    E = edge_src.shape[0]
    DE = edge_attr.shape[1]
    NB = edge_length_embedded.shape[1]
    RN = W_e1.shape[0]

    NS = 16
    NC = 2
    CH = 128
    H = N // NC
    NCH = (E + NS * CH - 1) // (NS * CH)
    ept = NCH * CH
    E_pad = NS * ept
    NSP = ((H + 1 + NS * 8 - 1) // (NS * 8)) * NS * 8

    fctp_scale = 1.0 / math.sqrt(D * A)
    c_s, c_x = math.sin(math.pi / 8), math.cos(math.pi / 8)

    pad = E_pad - E
    src_p = jnp.pad(edge_src.astype(jnp.int32), (0, pad)).reshape(NS, NCH, CH)
    dst_p = jnp.pad(edge_dst.astype(jnp.int32), (0, pad),
                    constant_values=N).reshape(NS, NCH, CH)
    W_in_r = jnp.transpose(W_in, (1, 0, 2))
    W_mask_r = jnp.transpose(W_mask, (1, 0, 2))
    W_out_r = jnp.transpose(W_out, (1, 0, 2))
    W2z = jnp.transpose(W_e2.reshape(RN, D, DE), (2, 0, 1)).reshape(DE * RN, D)
    W_e0b = W_e0.astype(jnp.bfloat16)
    W_e1b = W_e1.astype(jnp.bfloat16)
    W2zb = W2z.astype(jnp.bfloat16)

    BN = 1000
    grid_n = N // BN
    nf, mask = pl.pallas_call(
        functools.partial(_node_fctp_body, fctp_scale),
        grid=(grid_n,),
        in_specs=[
            pl.BlockSpec((BN, D), lambda i: (i, 0)),
            pl.BlockSpec((BN, A), lambda i: (i, 0)),
            pl.BlockSpec((BN, 1), lambda i: (i, 0)),
            pl.BlockSpec((A, D, D), lambda i: (0, 0, 0)),
            pl.BlockSpec((A, D, D), lambda i: (0, 0, 0)),
        ],
        out_specs=[
            pl.BlockSpec((BN, D), lambda i: (i, 0)),
            pl.BlockSpec((BN, D), lambda i: (i, 0)),
        ],
        out_shape=[
            jax.ShapeDtypeStruct((N, D), jnp.float32),
            jax.ShapeDtypeStruct((N, D), jnp.float32),
        ],
    )(node_input, node_attr, node_deg, W_in_r, W_mask_r)

    BE = 2048
    grid_e = E_pad // BE
    LB = (E + BE - 1) // BE - 1
    s0 = 1.0 / math.sqrt(NB)
    s1 = 1.0 / math.sqrt(RN)
    s2 = 1.0 / (math.sqrt(RN) * math.sqrt(DE))
    s_dense = pl.pallas_call(
        functools.partial(_edge_mlp_body, s0, s1, s2, DE),
        grid=(grid_e,),
        in_specs=[
            pl.BlockSpec((BE, NB), lambda i: (jnp.minimum(i, LB), 0)),
            pl.BlockSpec((BE, DE), lambda i: (jnp.minimum(i, LB), 0)),
            pl.BlockSpec((NB, RN), lambda i: (0, 0)),
            pl.BlockSpec((RN, RN), lambda i: (0, 0)),
            pl.BlockSpec((DE * RN, D), lambda i: (0, 0)),
        ],
        out_specs=pl.BlockSpec((BE, D), lambda i: (i, 0)),
        out_shape=jax.ShapeDtypeStruct((E_pad, D), jnp.float32),
    )(edge_length_embedded, edge_attr, W_e0b, W_e1b, W2zb)

    sc_edge = functools.partial(
        pl.kernel,
        mesh=plsc.VectorSubcoreMesh(core_axis_name="c", subcore_axis_name="s"),
        out_type=jax.ShapeDtypeStruct((NC, NSP, D), jnp.float32),
        scratch_types=[
            pltpu.VMEM((NCH, CH), jnp.int32),
            pltpu.VMEM((NCH, CH), jnp.int32),
            pltpu.VMEM((CH, D), jnp.float32),
            pltpu.VMEM((CH, D), jnp.float32),
            pltpu.VMEM((1, CH), jnp.int32),
            pltpu.VMEM_SHARED((NSP, D), jnp.float32),
        ] + [pltpu.SemaphoreType.DMA],
    )(functools.partial(_sc_edge_body, N, D, CH, NCH, NSP, NC, H))
    aggp = sc_edge(nf, s_dense, src_p, dst_p)

    out = pl.pallas_call(
        functools.partial(_out_fctp_body, fctp_scale, c_s, c_x),
        grid=(grid_n,),
        in_specs=[
            pl.BlockSpec((1, BN, D), lambda i: (i // (H // BN), i % (H // BN), 0)),
            pl.BlockSpec((BN, A), lambda i: (i, 0)),
            pl.BlockSpec((BN, 1), lambda i: (i, 0)),
            pl.BlockSpec((BN, D), lambda i: (i, 0)),
            pl.BlockSpec((A, D, D), lambda i: (0, 0, 0)),
        ],
        out_specs=pl.BlockSpec((BN, D), lambda i: (i, 0)),
        out_shape=jax.ShapeDtypeStruct((N, D), jnp.float32),
    )(aggp, node_attr, node_deg, mask, W_out_r)
    return out


# speedup vs baseline: 1.3566x; 1.0682x over previous
import functools
import math

import jax
import jax.numpy as jnp
from jax import lax
from jax.experimental import pallas as pl
from jax.experimental.pallas import tpu as pltpu
from jax.experimental.pallas import tpu_sc as plsc


def _silu(x):
    return x * (1.0 / (1.0 + jnp.exp(-x)))


def _node_fctp_body(scale, x_ref, y_ref, deg_ref, wi_ref, wm_ref, nf_ref, mask_ref):
    x = x_ref[...]
    y = y_ref[...]
    bn, d = x.shape
    a = y.shape[1]
    accf = jnp.zeros((bn, d), jnp.float32)
    accm = jnp.zeros((bn, d), jnp.float32)
    for j in range(a):
        xy = x * y[:, j:j + 1]
        accf += jnp.dot(xy, wi_ref[j], preferred_element_type=jnp.float32)
        accm += jnp.dot(xy, wm_ref[j], preferred_element_type=jnp.float32)
    nf_ref[...] = accf * scale * lax.rsqrt(deg_ref[...])
    mask_ref[...] = accm * scale


def _edge_mlp_body(s0, s1, s2, de, elen_ref, attr_ref, w0_ref, w1_ref, w2_ref, s_ref):
    bf = jnp.bfloat16
    h = _silu(jnp.dot(elen_ref[...].astype(bf), w0_ref[...],
                      preferred_element_type=jnp.float32) * s0)
    h = _silu(jnp.dot(h.astype(bf), w1_ref[...],
                      preferred_element_type=jnp.float32) * s1)
    hb = h.astype(bf)
    attr = attr_ref[...].astype(bf)
    z = jnp.concatenate([hb * attr[:, v:v + 1] for v in range(de)], axis=1)
    s_ref[...] = jnp.dot(z, w2_ref[...], preferred_element_type=jnp.float32) * s2


def _out_fctp_body(scale, c_s, c_x, aggp_ref, y_ref, deg_ref, mask_ref, wo_ref, out_ref):
    agg = aggp_ref[0] * lax.rsqrt(deg_ref[...])
    y = y_ref[...]
    bn, d = agg.shape
    a = y.shape[1]
    acc = jnp.zeros((bn, d), jnp.float32)
    for j in range(a):
        acc += jnp.dot(agg * y[:, j:j + 1], wo_ref[j], preferred_element_type=jnp.float32)
    out_ref[...] = c_s * mask_ref[...] + c_x * scale * acc


def _sc_edge_body(N, D, CH, NCH, NSP, NC, H,
                  nf_hbm, s_hbm, src_hbm, dst_hbm, out_hbm,
                  src_v, dst_v, x_v, ef_v, dstl_v, agg_sh, sem):
    c = lax.axis_index("c")
    sid = lax.axis_index("s")
    lo = c * H

    zrows = NSP // 16
    zero16 = jnp.zeros((16,), jnp.float32)

    def zrow(r, carry):
        for cc in range(D // 16):
            ef_v[r, pl.ds(cc * 16, 16)] = zero16
        return carry

    lax.fori_loop(0, CH, zrow, 0)
    nzc, rem = zrows // CH, zrows % CH
    for t in range(nzc):
        pltpu.sync_copy(ef_v, agg_sh.at[pl.ds(sid * zrows + t * CH, CH)])
    if rem:
        pltpu.sync_copy(ef_v.at[pl.ds(0, rem)],
                        agg_sh.at[pl.ds(sid * zrows + nzc * CH, rem)])
    plsc.subcore_barrier()

    pltpu.sync_copy(src_hbm.at[sid], src_v)
    pltpu.sync_copy(dst_hbm.at[sid], dst_v)

    ept = NCH * CH
    base = sid * ept

    def chunk(j, carry):
        cp = pltpu.make_async_copy(nf_hbm.at[src_v.at[j]], x_v, sem)
        cp.start()
        pltpu.sync_copy(s_hbm.at[pl.ds(base + j * CH, CH)], ef_v)
        cp.wait()

        def mrow(r, cr):
            for cc in range(D // 16):
                sl = pl.ds(cc * 16, 16)
                ef_v[r, sl] = ef_v[r, sl] * x_v[r, sl]
            return cr

        lax.fori_loop(0, CH, mrow, 0)
        for cc in range(CH // 16):
            sl = pl.ds(cc * 16, 16)
            loc = dst_v[j, sl] - lo
            ok = (loc >= 0) & (loc < H)
            dstl_v[0, sl] = jnp.where(ok, loc, H)
        pltpu.sync_copy(ef_v, agg_sh.at[dstl_v.at[0]], add=True)
        return carry

    lax.fori_loop(0, NCH, chunk, 0)
    plsc.subcore_barrier()

    orows = NSP // 16
    pltpu.sync_copy(agg_sh.at[pl.ds(sid * orows, orows)],
                    out_hbm.at[c].at[pl.ds(sid * orows, orows)])


def kernel(node_input, node_attr, node_deg, edge_src, edge_dst, edge_attr,
           edge_length_embedded, numb, n, W_in, W_mask, W_out, W_e0, W_e1, W_e2):
    N, D = node_input.shape
    A = node_attr.shape[1]
    E = edge_src.shape[0]
    DE = edge_attr.shape[1]
    NB = edge_length_embedded.shape[1]
    RN = W_e1.shape[0]

    NS = 16
    NC = 2
    CH = 128
    H = N // NC
    NCH = (E + NS * CH - 1) // (NS * CH)
    ept = NCH * CH
    E_pad = NS * ept
    NSP = ((H + 1 + NS * 8 - 1) // (NS * 8)) * NS * 8

    fctp_scale = 1.0 / math.sqrt(D * A)
    c_s, c_x = math.sin(math.pi / 8), math.cos(math.pi / 8)

    pad = E_pad - E
    src_p = jnp.pad(edge_src.astype(jnp.int32), (0, pad)).reshape(NS, NCH, CH)
    dst_p = jnp.pad(edge_dst.astype(jnp.int32), (0, pad),
                    constant_values=N).reshape(NS, NCH, CH)
    W_in_r = jnp.transpose(W_in, (1, 0, 2))
    W_mask_r = jnp.transpose(W_mask, (1, 0, 2))
    W_out_r = jnp.transpose(W_out, (1, 0, 2))
    W2z = jnp.transpose(W_e2.reshape(RN, D, DE), (2, 0, 1)).reshape(DE * RN, D)
    W_e0b = W_e0.astype(jnp.bfloat16)
    W_e1b = W_e1.astype(jnp.bfloat16)
    W2zb = W2z.astype(jnp.bfloat16)

    BN = 1000
    grid_n = N // BN
    nf, mask = pl.pallas_call(
        functools.partial(_node_fctp_body, fctp_scale),
        grid=(grid_n,),
        in_specs=[
            pl.BlockSpec((BN, D), lambda i: (i, 0)),
            pl.BlockSpec((BN, A), lambda i: (i, 0)),
            pl.BlockSpec((BN, 1), lambda i: (i, 0)),
            pl.BlockSpec((A, D, D), lambda i: (0, 0, 0)),
            pl.BlockSpec((A, D, D), lambda i: (0, 0, 0)),
        ],
        out_specs=[
            pl.BlockSpec((BN, D), lambda i: (i, 0)),
            pl.BlockSpec((BN, D), lambda i: (i, 0)),
        ],
        out_shape=[
            jax.ShapeDtypeStruct((N, D), jnp.float32),
            jax.ShapeDtypeStruct((N, D), jnp.float32),
        ],
    )(node_input, node_attr, node_deg, W_in_r, W_mask_r)

    BE = 2048
    grid_e = E_pad // BE
    LB = (E + BE - 1) // BE - 1
    s0 = 1.0 / math.sqrt(NB)
    s1 = 1.0 / math.sqrt(RN)
    s2 = 1.0 / (math.sqrt(RN) * math.sqrt(DE))
    s_dense = pl.pallas_call(
        functools.partial(_edge_mlp_body, s0, s1, s2, DE),
        grid=(grid_e,),
        in_specs=[
            pl.BlockSpec((BE, NB), lambda i: (jnp.minimum(i, LB), 0)),
            pl.BlockSpec((BE, DE), lambda i: (jnp.minimum(i, LB), 0)),
            pl.BlockSpec((NB, RN), lambda i: (0, 0)),
            pl.BlockSpec((RN, RN), lambda i: (0, 0)),
            pl.BlockSpec((DE * RN, D), lambda i: (0, 0)),
        ],
        out_specs=pl.BlockSpec((BE, D), lambda i: (i, 0)),
        out_shape=jax.ShapeDtypeStruct((E_pad, D), jnp.float32),
    )(edge_length_embedded, edge_attr, W_e0b, W_e1b, W2zb)

    sc_edge = functools.partial(
        pl.kernel,
        mesh=plsc.VectorSubcoreMesh(core_axis_name="c", subcore_axis_name="s"),
        out_type=jax.ShapeDtypeStruct((NC, NSP, D), jnp.float32),
        scratch_types=[
            pltpu.VMEM((NCH, CH), jnp.int32),
            pltpu.VMEM((NCH, CH), jnp.int32),
            pltpu.VMEM((CH, D), jnp.float32),
            pltpu.VMEM((CH, D), jnp.float32),
            pltpu.VMEM((1, CH), jnp.int32),
            pltpu.VMEM_SHARED((NSP, D), jnp.float32),
        ] + [pltpu.SemaphoreType.DMA],
    )(functools.partial(_sc_edge_body, N, D, CH, NCH, NSP, NC, H))
    aggp = sc_edge(nf, s_dense, src_p, dst_p)

    out = pl.pallas_call(
        functools.partial(_out_fctp_body, fctp_scale, c_s, c_x),
        grid=(grid_n,),
        in_specs=[
            pl.BlockSpec((1, BN, D), lambda i: (i // (H // BN), i % (H // BN), 0)),
            pl.BlockSpec((BN, A), lambda i: (i, 0)),
            pl.BlockSpec((BN, 1), lambda i: (i, 0)),
            pl.BlockSpec((BN, D), lambda i: (i, 0)),
            pl.BlockSpec((A, D, D), lambda i: (0, 0, 0)),
        ],
        out_specs=pl.BlockSpec((BN, D), lambda i: (i, 0)),
        out_shape=jax.ShapeDtypeStruct((N, D), jnp.float32),
    )(aggp, node_attr, node_deg, mask, W_out_r)
    return out


# speedup vs baseline: 1.4230x; 1.0490x over previous
import functools
import math

import jax
import jax.numpy as jnp
from jax import lax
from jax.experimental import pallas as pl
from jax.experimental.pallas import tpu as pltpu
from jax.experimental.pallas import tpu_sc as plsc


def _silu(x):
    return x * (1.0 / (1.0 + jnp.exp(-x)))


def _node_fctp_body(scale, x_ref, y_ref, deg_ref, wi_ref, wm_ref, nf_ref, mask_ref):
    x = x_ref[...]
    y = y_ref[...]
    bn, d = x.shape
    a = y.shape[1]
    accf = jnp.zeros((bn, d), jnp.float32)
    accm = jnp.zeros((bn, d), jnp.float32)
    for j in range(a):
        xy = x * y[:, j:j + 1]
        accf += jnp.dot(xy, wi_ref[j], preferred_element_type=jnp.float32)
        accm += jnp.dot(xy, wm_ref[j], preferred_element_type=jnp.float32)
    nf_ref[...] = accf * scale * lax.rsqrt(deg_ref[...])
    mask_ref[...] = accm * scale


def _edge_mlp_body(s0, s1, s2, de, elen_ref, attr_ref, w0_ref, w1_ref, w2_ref, s_ref):
    bf = jnp.bfloat16
    h = _silu(jnp.dot(elen_ref[...].astype(bf), w0_ref[...],
                      preferred_element_type=jnp.float32) * s0)
    h = _silu(jnp.dot(h.astype(bf), w1_ref[...],
                      preferred_element_type=jnp.float32) * s1)
    hb = h.astype(bf)
    attr = attr_ref[...].astype(bf)
    z = jnp.concatenate([hb * attr[:, v:v + 1] for v in range(de)], axis=1)
    s_ref[...] = jnp.dot(z, w2_ref[...], preferred_element_type=jnp.float32) * s2


def _out_fctp_body(scale, c_s, c_x, aggp_ref, y_ref, deg_ref, mask_ref, wo_ref, out_ref):
    agg = aggp_ref[0] * lax.rsqrt(deg_ref[...])
    y = y_ref[...]
    bn, d = agg.shape
    a = y.shape[1]
    acc = jnp.zeros((bn, d), jnp.float32)
    for j in range(a):
        acc += jnp.dot(agg * y[:, j:j + 1], wo_ref[j], preferred_element_type=jnp.float32)
    out_ref[...] = c_s * mask_ref[...] + c_x * scale * acc


def _sc_edge_body(N, D, CH, NCH, NSP, NC, H,
                  nf_hbm, s_hbm, pk_hbm, out_hbm,
                  pk_v, srcl_v, x_v, ef_v, dstl_v, agg_sh, sem, c0, c1):
    c = lax.axis_index("c")
    sid = lax.axis_index("s")
    lo = c * H
    csems = [c0, c1]

    zrows = NSP // 16
    zero16 = jnp.zeros((16,), jnp.float32)

    def zrow(r, carry):
        for cc in range(D // 16):
            ef_v[0, r, pl.ds(cc * 16, 16)] = zero16
        return carry

    lax.fori_loop(0, CH, zrow, 0)
    nzc, rem = zrows // CH, zrows % CH
    for t in range(nzc):
        pltpu.sync_copy(ef_v.at[0], agg_sh.at[pl.ds(sid * zrows + t * CH, CH)])
    if rem:
        pltpu.sync_copy(ef_v.at[0].at[pl.ds(0, rem)],
                        agg_sh.at[pl.ds(sid * zrows + nzc * CH, rem)])
    plsc.subcore_barrier()

    pltpu.sync_copy(pk_hbm.at[sid], pk_v)

    ept = NCH * CH
    base = sid * ept

    def wait_scat(b):
        pltpu.make_async_copy(ef_v.at[b], agg_sh.at[dstl_v.at[b]],
                              csems[b]).wait()

    def do_chunk(j, b, first):
        for cc in range(CH // 16):
            sl = pl.ds(cc * 16, 16)
            p16 = pk_v[j, sl]
            srcl_v[0, sl] = jnp.bitwise_and(p16, 16383)
            loc = jnp.right_shift(p16, 14) - lo
            ok = (loc >= 0) & (loc < H)
            dstl_v[b, sl] = jnp.where(ok, loc, H)
        cp = pltpu.make_async_copy(nf_hbm.at[srcl_v.at[0]], x_v, sem)
        cp.start()
        if not first:
            wait_scat(b)
        pltpu.sync_copy(s_hbm.at[pl.ds(base + j * CH, CH)], ef_v.at[b])
        cp.wait()

        def mrow(r, cr):
            for cc in range(D // 16):
                sl = pl.ds(cc * 16, 16)
                ef_v[b, r, sl] = ef_v[b, r, sl] * x_v[r, sl]
            return cr

        lax.fori_loop(0, CH, mrow, 0)
        pltpu.async_copy(ef_v.at[b], agg_sh.at[dstl_v.at[b]], csems[b],
                         add=True)

    do_chunk(0, 0, True)
    do_chunk(1, 1, True)
    do_chunk(2, 0, False)

    def pair(k, carry):
        do_chunk(3 + 2 * k, 1, False)
        do_chunk(4 + 2 * k, 0, False)
        return carry

    lax.fori_loop(0, (NCH - 3) // 2, pair, 0)
    wait_scat(0)
    wait_scat(1)
    plsc.subcore_barrier()

    orows = NSP // 16
    pltpu.sync_copy(agg_sh.at[pl.ds(sid * orows, orows)],
                    out_hbm.at[c].at[pl.ds(sid * orows, orows)])


def kernel(node_input, node_attr, node_deg, edge_src, edge_dst, edge_attr,
           edge_length_embedded, numb, n, W_in, W_mask, W_out, W_e0, W_e1, W_e2):
    N, D = node_input.shape
    A = node_attr.shape[1]
    E = edge_src.shape[0]
    DE = edge_attr.shape[1]
    NB = edge_length_embedded.shape[1]
    RN = W_e1.shape[0]

    NS = 16
    NC = 2
    CH = 128
    H = N // NC
    NCH = (E + NS * CH - 1) // (NS * CH)
    assert NCH % 2 == 1
    ept = NCH * CH
    E_pad = NS * ept
    NSP = ((H + 1 + NS * 8 - 1) // (NS * 8)) * NS * 8

    fctp_scale = 1.0 / math.sqrt(D * A)
    c_s, c_x = math.sin(math.pi / 8), math.cos(math.pi / 8)

    pad = E_pad - E
    src_p = jnp.pad(edge_src.astype(jnp.int32), (0, pad))
    dst_p = jnp.pad(edge_dst.astype(jnp.int32), (0, pad), constant_values=N)
    pk_p = (src_p + dst_p * 16384).reshape(NS, NCH, CH)
    W_in_r = jnp.transpose(W_in, (1, 0, 2))
    W_mask_r = jnp.transpose(W_mask, (1, 0, 2))
    W_out_r = jnp.transpose(W_out, (1, 0, 2))
    W2z = jnp.transpose(W_e2.reshape(RN, D, DE), (2, 0, 1)).reshape(DE * RN, D)
    W_e0b = W_e0.astype(jnp.bfloat16)
    W_e1b = W_e1.astype(jnp.bfloat16)
    W2zb = W2z.astype(jnp.bfloat16)

    BN = 1000
    grid_n = N // BN
    nf, mask = pl.pallas_call(
        functools.partial(_node_fctp_body, fctp_scale),
        grid=(grid_n,),
        in_specs=[
            pl.BlockSpec((BN, D), lambda i: (i, 0)),
            pl.BlockSpec((BN, A), lambda i: (i, 0)),
            pl.BlockSpec((BN, 1), lambda i: (i, 0)),
            pl.BlockSpec((A, D, D), lambda i: (0, 0, 0)),
            pl.BlockSpec((A, D, D), lambda i: (0, 0, 0)),
        ],
        out_specs=[
            pl.BlockSpec((BN, D), lambda i: (i, 0)),
            pl.BlockSpec((BN, D), lambda i: (i, 0)),
        ],
        out_shape=[
            jax.ShapeDtypeStruct((N, D), jnp.float32),
            jax.ShapeDtypeStruct((N, D), jnp.float32),
        ],
    )(node_input, node_attr, node_deg, W_in_r, W_mask_r)

    BE = 2048
    grid_e = E_pad // BE
    LB = (E + BE - 1) // BE - 1
    s0 = 1.0 / math.sqrt(NB)
    s1 = 1.0 / math.sqrt(RN)
    s2 = 1.0 / (math.sqrt(RN) * math.sqrt(DE))
    s_dense = pl.pallas_call(
        functools.partial(_edge_mlp_body, s0, s1, s2, DE),
        grid=(grid_e,),
        in_specs=[
            pl.BlockSpec((BE, NB), lambda i: (jnp.minimum(i, LB), 0)),
            pl.BlockSpec((BE, DE), lambda i: (jnp.minimum(i, LB), 0)),
            pl.BlockSpec((NB, RN), lambda i: (0, 0)),
            pl.BlockSpec((RN, RN), lambda i: (0, 0)),
            pl.BlockSpec((DE * RN, D), lambda i: (0, 0)),
        ],
        out_specs=pl.BlockSpec((BE, D), lambda i: (i, 0)),
        out_shape=jax.ShapeDtypeStruct((E_pad, D), jnp.float32),
    )(edge_length_embedded, edge_attr, W_e0b, W_e1b, W2zb)

    sc_edge = functools.partial(
        pl.kernel,
        mesh=plsc.VectorSubcoreMesh(core_axis_name="c", subcore_axis_name="s"),
        out_type=jax.ShapeDtypeStruct((NC, NSP, D), jnp.float32),
        scratch_types=[
            pltpu.VMEM((NCH, CH), jnp.int32),
            pltpu.VMEM((1, CH), jnp.int32),
            pltpu.VMEM((CH, D), jnp.float32),
            pltpu.VMEM((2, CH, D), jnp.float32),
            pltpu.VMEM((2, CH), jnp.int32),
            pltpu.VMEM_SHARED((NSP, D), jnp.float32),
        ] + [pltpu.SemaphoreType.DMA] * 3,
    )(functools.partial(_sc_edge_body, N, D, CH, NCH, NSP, NC, H))
    aggp = sc_edge(nf, s_dense, pk_p)

    out = pl.pallas_call(
        functools.partial(_out_fctp_body, fctp_scale, c_s, c_x),
        grid=(grid_n,),
        in_specs=[
            pl.BlockSpec((1, BN, D), lambda i: (i // (H // BN), i % (H // BN), 0)),
            pl.BlockSpec((BN, A), lambda i: (i, 0)),
            pl.BlockSpec((BN, 1), lambda i: (i, 0)),
            pl.BlockSpec((BN, D), lambda i: (i, 0)),
            pl.BlockSpec((A, D, D), lambda i: (0, 0, 0)),
        ],
        out_specs=pl.BlockSpec((BN, D), lambda i: (i, 0)),
        out_shape=jax.ShapeDtypeStruct((N, D), jnp.float32),
    )(aggp, node_attr, node_deg, mask, W_out_r)
    return out
